# baseline (speedup 1.0000x reference)
import functools

import jax
import jax.numpy as jnp
from jax.experimental import pallas as pl
from jax.experimental.pallas import tpu as pltpu

_BETA = 0.25


def _rqvae_kernel(
    x_ref, w0_ref, b0_ref, w1_ref, b1_ref, w2_ref, b2_ref,
    dw0_ref, db0_ref, dw1_ref, db1_ref, dw2_ref, db2_ref, cb_ref,
    out_ref, idx_ref, loss_ref, *, n_codebooks, batch_total):
    x = x_ref[...]
    h = jax.nn.relu(jnp.dot(x, w0_ref[...], preferred_element_type=jnp.float32)
                    + b0_ref[...])
    h = jax.nn.relu(jnp.dot(h, w1_ref[...], preferred_element_type=jnp.float32)
                    + b1_ref[...])
    z = jnp.dot(h, w2_ref[...], preferred_element_type=jnp.float32) + b2_ref[...]

    blk = z.shape[0]
    k = cb_ref.shape[1]
    residual = z
    quant = jnp.zeros_like(z)
    loss_sum = jnp.float32(0.0)
    idx_cols = []
    lane_ids = jax.lax.broadcasted_iota(jnp.int32, (blk, k), 1)
    for i in range(n_codebooks):
        cb = cb_ref[i]
        cb_norm = jnp.sum(cb * cb, axis=1)
        d = (jnp.sum(residual * residual, axis=1, keepdims=True)
             - 2.0 * jnp.dot(residual, cb.T, preferred_element_type=jnp.float32)
             + cb_norm[None, :])
        idx = jnp.argmin(d, axis=1).astype(jnp.int32)
        one_hot = (lane_ids == idx[:, None]).astype(jnp.float32)
        e = jnp.dot(one_hot, cb, preferred_element_type=jnp.float32,
                    precision=jax.lax.Precision.HIGHEST)
        residual = residual - e
        loss_sum = loss_sum + jnp.sum(residual * residual)
        quant = quant + e
        idx_cols.append(idx[:, None])

    r = jax.nn.relu(jnp.dot(quant, dw0_ref[...], preferred_element_type=jnp.float32)
                    + db0_ref[...])
    r = jax.nn.relu(jnp.dot(r, dw1_ref[...], preferred_element_type=jnp.float32)
                    + db1_ref[...])
    out_ref[...] = (jnp.dot(r, dw2_ref[...], preferred_element_type=jnp.float32)
                    + db2_ref[...])
    idx_ref[...] = jnp.concatenate(idx_cols, axis=1)

    @pl.when(pl.program_id(0) == 0)
    def _():
        loss_ref[...] = jnp.zeros_like(loss_ref)

    loss_ref[...] += loss_sum * ((1.0 + _BETA) / batch_total)


def kernel(embeddings, enc_w0, enc_b0, enc_w1, enc_b1, enc_w2, enc_b2,
           dec_w0, dec_b0, dec_w1, dec_b1, dec_w2, dec_b2, codebooks):
    b_total, d_in = embeddings.shape
    ncb, k, e_dim = codebooks.shape
    blk = 512 if b_total % 512 == 0 else b_total
    grid = b_total // blk

    full = lambda shape: pl.BlockSpec(shape, lambda i: (0,) * len(shape))
    row2 = lambda v: v.reshape(1, -1)

    kern = functools.partial(_rqvae_kernel, n_codebooks=ncb,
                             batch_total=float(b_total))
    out_shapes = (
        jax.ShapeDtypeStruct((b_total, d_in), jnp.float32),
        jax.ShapeDtypeStruct((b_total, ncb), jnp.int32),
        jax.ShapeDtypeStruct((1, 1), jnp.float32),
    )
    recon, idx, loss = pl.pallas_call(
        kern,
        grid=(grid,),
        in_specs=[
            pl.BlockSpec((blk, d_in), lambda i: (i, 0)),
            full(enc_w0.shape), full((1, enc_b0.shape[0])),
            full(enc_w1.shape), full((1, enc_b1.shape[0])),
            full(enc_w2.shape), full((1, enc_b2.shape[0])),
            full(dec_w0.shape), full((1, dec_b0.shape[0])),
            full(dec_w1.shape), full((1, dec_b1.shape[0])),
            full(dec_w2.shape), full((1, dec_b2.shape[0])),
            full(codebooks.shape),
        ],
        out_specs=(
            pl.BlockSpec((blk, d_in), lambda i: (i, 0)),
            pl.BlockSpec((blk, ncb), lambda i: (i, 0)),
            pl.BlockSpec((1, 1), lambda i: (0, 0)),
        ),
        out_shape=out_shapes,
        compiler_params=pltpu.CompilerParams(
            dimension_semantics=("arbitrary",)),
    )(embeddings, enc_w0, row2(enc_b0), enc_w1, row2(enc_b1),
      enc_w2, row2(enc_b2), dec_w0, row2(dec_b0), dec_w1, row2(dec_b1),
      dec_w2, row2(dec_b2), codebooks)
    return recon, idx.astype(jnp.int64), loss[0, 0]


# speedup vs baseline: 1.4259x; 1.4259x over previous
import functools

import jax
import jax.numpy as jnp
from jax.experimental import pallas as pl
from jax.experimental.pallas import tpu as pltpu

_BETA = 0.25


def _rqvae_kernel(
    x_ref, w0_ref, b0_ref, w1_ref, b1_ref, w2_ref, b2_ref,
    dw0_ref, db0_ref, dw1_ref, db1_ref, dw2_ref, db2_ref,
    cb_ref, cbt_ref, cbh_ref, cbm_ref, cbl_ref,
    out_ref, idx_ref, loss_ref, *, n_codebooks, batch_total):
    bf = jnp.bfloat16
    f32 = jnp.float32
    dot = lambda a, b: jnp.dot(a, b, preferred_element_type=f32)

    x = x_ref[...]
    h = jax.nn.relu(dot(x.astype(bf), w0_ref[...]) + b0_ref[...])
    h = jax.nn.relu(dot(h.astype(bf), w1_ref[...]) + b1_ref[...])
    z = dot(h.astype(bf), w2_ref[...]) + b2_ref[...]

    blk = z.shape[0]
    k = cb_ref.shape[1]
    residual = z
    quant = jnp.zeros_like(z)
    loss_sum = jnp.float32(0.0)
    idx_cols = []
    lane_ids = jax.lax.broadcasted_iota(jnp.int32, (blk, k), 1)
    for i in range(n_codebooks):
        cb = cb_ref[i]
        cb_norm = jnp.sum(cb * cb, axis=1)
        r_norm = jnp.sum(residual * residual, axis=1, keepdims=True)
        d = (r_norm - 2.0 * dot(residual.astype(bf), cbt_ref[i])) + cb_norm[None, :]
        idx = jnp.argmin(d, axis=1).astype(jnp.int32)
        one_hot = (lane_ids == idx[:, None]).astype(bf)
        e = ((dot(one_hot, cbh_ref[i]) + dot(one_hot, cbm_ref[i]))
             + dot(one_hot, cbl_ref[i]))
        residual = residual - e
        loss_sum = loss_sum + jnp.sum(residual * residual)
        quant = quant + e
        idx_cols.append(idx[:, None])

    r = jax.nn.relu(dot(quant.astype(bf), dw0_ref[...]) + db0_ref[...])
    r = jax.nn.relu(dot(r.astype(bf), dw1_ref[...]) + db1_ref[...])
    out_ref[...] = dot(r.astype(bf), dw2_ref[...]) + db2_ref[...]
    idx_ref[...] = jnp.concatenate(idx_cols, axis=1)

    @pl.when(pl.program_id(0) == 0)
    def _():
        loss_ref[...] = jnp.zeros_like(loss_ref)

    loss_ref[...] += loss_sum * ((1.0 + _BETA) / batch_total)


def kernel(embeddings, enc_w0, enc_b0, enc_w1, enc_b1, enc_w2, enc_b2,
           dec_w0, dec_b0, dec_w1, dec_b1, dec_w2, dec_b2, codebooks):
    b_total, d_in = embeddings.shape
    ncb, k, e_dim = codebooks.shape
    blk = 512 if b_total % 512 == 0 else b_total
    grid = b_total // blk

    bf = jnp.bfloat16
    f32 = jnp.float32
    cb_hi = codebooks.astype(bf)
    rem = codebooks - cb_hi.astype(f32)
    cb_mid = rem.astype(bf)
    cb_lo = (rem - cb_mid.astype(f32)).astype(bf)
    cb_t = jnp.swapaxes(cb_hi, 1, 2)

    full = lambda shape: pl.BlockSpec(shape, lambda i: (0,) * len(shape))
    row2 = lambda v: v.reshape(1, -1)

    kern = functools.partial(_rqvae_kernel, n_codebooks=ncb,
                             batch_total=float(b_total))
    out_shapes = (
        jax.ShapeDtypeStruct((b_total, d_in), jnp.float32),
        jax.ShapeDtypeStruct((b_total, ncb), jnp.int32),
        jax.ShapeDtypeStruct((1, 1), jnp.float32),
    )
    recon, idx, loss = pl.pallas_call(
        kern,
        grid=(grid,),
        in_specs=[
            pl.BlockSpec((blk, d_in), lambda i: (i, 0)),
            full(enc_w0.shape), full((1, enc_b0.shape[0])),
            full(enc_w1.shape), full((1, enc_b1.shape[0])),
            full(enc_w2.shape), full((1, enc_b2.shape[0])),
            full(dec_w0.shape), full((1, dec_b0.shape[0])),
            full(dec_w1.shape), full((1, dec_b1.shape[0])),
            full(dec_w2.shape), full((1, dec_b2.shape[0])),
            full(codebooks.shape), full((ncb, e_dim, k)),
            full(codebooks.shape), full(codebooks.shape), full(codebooks.shape),
        ],
        out_specs=(
            pl.BlockSpec((blk, d_in), lambda i: (i, 0)),
            pl.BlockSpec((blk, ncb), lambda i: (i, 0)),
            pl.BlockSpec((1, 1), lambda i: (0, 0)),
        ),
        out_shape=out_shapes,
        compiler_params=pltpu.CompilerParams(
            dimension_semantics=("arbitrary",)),
    )(embeddings, enc_w0.astype(bf), row2(enc_b0), enc_w1.astype(bf),
      row2(enc_b1), enc_w2.astype(bf), row2(enc_b2), dec_w0.astype(bf),
      row2(dec_b0), dec_w1.astype(bf), row2(dec_b1), dec_w2.astype(bf),
      row2(dec_b2), codebooks, cb_t, cb_hi, cb_mid, cb_lo)
    return recon, idx.astype(jnp.int64), loss[0, 0]


# speedup vs baseline: 1.4282x; 1.0016x over previous
import functools

import jax
import jax.numpy as jnp
from jax.experimental import pallas as pl
from jax.experimental.pallas import tpu as pltpu

_BETA = 0.25


def _rqvae_kernel(
    x_ref, w0_ref, b0_ref, w1_ref, b1_ref, w2_ref, b2_ref,
    dw0_ref, db0_ref, dw1_ref, db1_ref, dw2_ref, db2_ref,
    cbt_ref, cbh_ref, cbm_ref, cbl_ref, cbn_ref,
    out_ref, idx_ref, loss_ref, *, n_codebooks, batch_total):
    bf = jnp.bfloat16
    f32 = jnp.float32
    dot = lambda a, b: jnp.dot(a, b, preferred_element_type=f32)

    x = x_ref[...]
    h = jax.nn.relu(dot(x.astype(bf), w0_ref[...]) + b0_ref[...])
    h = jax.nn.relu(dot(h.astype(bf), w1_ref[...]) + b1_ref[...])
    z = dot(h.astype(bf), w2_ref[...]) + b2_ref[...]

    blk = z.shape[0]
    k = cbt_ref.shape[2]
    residual = z
    quant = jnp.zeros_like(z)
    loss_sum = jnp.float32(0.0)
    idx_cols = []
    lane_ids = jax.lax.broadcasted_iota(jnp.int32, (blk, k), 1)
    for i in range(n_codebooks):
        score = dot(residual.astype(bf), cbt_ref[i]) + cbn_ref[i]
        idx = jnp.argmin(score, axis=1).astype(jnp.int32)
        one_hot = (lane_ids == idx[:, None]).astype(bf)
        e = ((dot(one_hot, cbh_ref[i]) + dot(one_hot, cbm_ref[i]))
             + dot(one_hot, cbl_ref[i]))
        residual = residual - e
        loss_sum = loss_sum + jnp.sum(residual * residual)
        quant = quant + e
        idx_cols.append(idx[:, None])

    r = jax.nn.relu(dot(quant.astype(bf), dw0_ref[...]) + db0_ref[...])
    r = jax.nn.relu(dot(r.astype(bf), dw1_ref[...]) + db1_ref[...])
    out_ref[...] = dot(r.astype(bf), dw2_ref[...]) + db2_ref[...]
    idx_ref[...] = jnp.concatenate(idx_cols, axis=1)

    @pl.when(pl.program_id(0) == 0)
    def _():
        loss_ref[...] = jnp.zeros_like(loss_ref)

    loss_ref[...] += loss_sum * ((1.0 + _BETA) / batch_total)


def kernel(embeddings, enc_w0, enc_b0, enc_w1, enc_b1, enc_w2, enc_b2,
           dec_w0, dec_b0, dec_w1, dec_b1, dec_w2, dec_b2, codebooks):
    b_total, d_in = embeddings.shape
    ncb, k, e_dim = codebooks.shape
    blk = 512 if b_total % 512 == 0 else b_total
    grid = b_total // blk

    bf = jnp.bfloat16
    f32 = jnp.float32
    cb_hi = codebooks.astype(bf)
    rem = codebooks - cb_hi.astype(f32)
    cb_mid = rem.astype(bf)
    cb_lo = (rem - cb_mid.astype(f32)).astype(bf)
    cb_t = jnp.swapaxes(cb_hi * jnp.bfloat16(-2.0), 1, 2)
    cb_norm = jnp.sum(codebooks * codebooks, axis=2)[:, None, :]

    full = lambda shape: pl.BlockSpec(shape, lambda i: (0,) * len(shape))
    row2 = lambda v: v.reshape(1, -1)

    kern = functools.partial(_rqvae_kernel, n_codebooks=ncb,
                             batch_total=float(b_total))
    out_shapes = (
        jax.ShapeDtypeStruct((b_total, d_in), jnp.float32),
        jax.ShapeDtypeStruct((b_total, ncb), jnp.int32),
        jax.ShapeDtypeStruct((1, 1), jnp.float32),
    )
    recon, idx, loss = pl.pallas_call(
        kern,
        grid=(grid,),
        in_specs=[
            pl.BlockSpec((blk, d_in), lambda i: (i, 0)),
            full(enc_w0.shape), full((1, enc_b0.shape[0])),
            full(enc_w1.shape), full((1, enc_b1.shape[0])),
            full(enc_w2.shape), full((1, enc_b2.shape[0])),
            full(dec_w0.shape), full((1, dec_b0.shape[0])),
            full(dec_w1.shape), full((1, dec_b1.shape[0])),
            full(dec_w2.shape), full((1, dec_b2.shape[0])),
            full((ncb, e_dim, k)),
            full(codebooks.shape), full(codebooks.shape), full(codebooks.shape),
            full((ncb, 1, k)),
        ],
        out_specs=(
            pl.BlockSpec((blk, d_in), lambda i: (i, 0)),
            pl.BlockSpec((blk, ncb), lambda i: (i, 0)),
            pl.BlockSpec((1, 1), lambda i: (0, 0)),
        ),
        out_shape=out_shapes,
        compiler_params=pltpu.CompilerParams(
            dimension_semantics=("arbitrary",)),
    )(embeddings, enc_w0.astype(bf), row2(enc_b0), enc_w1.astype(bf),
      row2(enc_b1), enc_w2.astype(bf), row2(enc_b2), dec_w0.astype(bf),
      row2(dec_b0), dec_w1.astype(bf), row2(dec_b1), dec_w2.astype(bf),
      row2(dec_b2), cb_t, cb_hi, cb_mid, cb_lo, cb_norm)
    return recon, idx.astype(jnp.int64), loss[0, 0]


# speedup vs baseline: 1.4840x; 1.0391x over previous
import functools

import jax
import jax.numpy as jnp
from jax.experimental import pallas as pl
from jax.experimental.pallas import tpu as pltpu

_BETA = 0.25


def _rqvae_kernel(
    x_ref, w0_ref, b0_ref, w1_ref, b1_ref, w2_ref, b2_ref,
    dw0_ref, db0_ref, dw1_ref, db1_ref, dw2_ref, db2_ref,
    cbt_ref, cbh_ref, cbm_ref, cbl_ref, cbn_ref,
    out_ref, idx_ref, loss_ref, *, n_codebooks, batch_total):
    bf = jnp.bfloat16
    f32 = jnp.float32
    dot = lambda a, b: jnp.dot(a, b, preferred_element_type=f32)

    def chain(x):
        h = jax.nn.relu(dot(x.astype(bf), w0_ref[...]) + b0_ref[...])
        h = jax.nn.relu(dot(h.astype(bf), w1_ref[...]) + b1_ref[...])
        z = dot(h.astype(bf), w2_ref[...]) + b2_ref[...]

        rows = z.shape[0]
        k = cbt_ref.shape[2]
        residual = z
        quant = jnp.zeros_like(z)
        loss_sum = jnp.float32(0.0)
        idx_cols = []
        lane_ids = jax.lax.broadcasted_iota(jnp.int32, (rows, k), 1)
        for i in range(n_codebooks):
            score = dot(residual.astype(bf), cbt_ref[i]) + cbn_ref[i]
            idx = jnp.argmin(score, axis=1).astype(jnp.int32)
            one_hot = (lane_ids == idx[:, None]).astype(bf)
            e = ((dot(one_hot, cbh_ref[i]) + dot(one_hot, cbm_ref[i]))
                 + dot(one_hot, cbl_ref[i]))
            residual = residual - e
            loss_sum = loss_sum + jnp.sum(residual * residual)
            quant = quant + e
            idx_cols.append(idx[:, None])

        r = jax.nn.relu(dot(quant.astype(bf), dw0_ref[...]) + db0_ref[...])
        r = jax.nn.relu(dot(r.astype(bf), dw1_ref[...]) + db1_ref[...])
        recon = dot(r.astype(bf), dw2_ref[...]) + db2_ref[...]
        return recon, jnp.concatenate(idx_cols, axis=1), loss_sum

    blk = x_ref.shape[0]
    half = blk // 2
    recon_a, idx_a, loss_a = chain(x_ref[0:half, :])
    recon_b, idx_b, loss_b = chain(x_ref[half:blk, :])
    out_ref[0:half, :] = recon_a
    out_ref[half:blk, :] = recon_b
    idx_ref[0:half, :] = idx_a
    idx_ref[half:blk, :] = idx_b

    @pl.when(pl.program_id(0) == 0)
    def _():
        loss_ref[...] = jnp.zeros_like(loss_ref)

    loss_ref[...] += (loss_a + loss_b) * ((1.0 + _BETA) / batch_total)


def kernel(embeddings, enc_w0, enc_b0, enc_w1, enc_b1, enc_w2, enc_b2,
           dec_w0, dec_b0, dec_w1, dec_b1, dec_w2, dec_b2, codebooks):
    b_total, d_in = embeddings.shape
    ncb, k, e_dim = codebooks.shape
    blk = 1024 if b_total % 1024 == 0 else b_total
    grid = b_total // blk

    bf = jnp.bfloat16
    f32 = jnp.float32
    cb_hi = codebooks.astype(bf)
    rem = codebooks - cb_hi.astype(f32)
    cb_mid = rem.astype(bf)
    cb_lo = (rem - cb_mid.astype(f32)).astype(bf)
    cb_t = jnp.swapaxes(cb_hi * jnp.bfloat16(-2.0), 1, 2)
    cb_norm = jnp.sum(codebooks * codebooks, axis=2)[:, None, :]

    full = lambda shape: pl.BlockSpec(shape, lambda i: (0,) * len(shape))
    row2 = lambda v: v.reshape(1, -1)

    kern = functools.partial(_rqvae_kernel, n_codebooks=ncb,
                             batch_total=float(b_total))
    out_shapes = (
        jax.ShapeDtypeStruct((b_total, d_in), jnp.float32),
        jax.ShapeDtypeStruct((b_total, ncb), jnp.int32),
        jax.ShapeDtypeStruct((1, 1), jnp.float32),
    )
    recon, idx, loss = pl.pallas_call(
        kern,
        grid=(grid,),
        in_specs=[
            pl.BlockSpec((blk, d_in), lambda i: (i, 0)),
            full(enc_w0.shape), full((1, enc_b0.shape[0])),
            full(enc_w1.shape), full((1, enc_b1.shape[0])),
            full(enc_w2.shape), full((1, enc_b2.shape[0])),
            full(dec_w0.shape), full((1, dec_b0.shape[0])),
            full(dec_w1.shape), full((1, dec_b1.shape[0])),
            full(dec_w2.shape), full((1, dec_b2.shape[0])),
            full((ncb, e_dim, k)),
            full(codebooks.shape), full(codebooks.shape), full(codebooks.shape),
            full((ncb, 1, k)),
        ],
        out_specs=(
            pl.BlockSpec((blk, d_in), lambda i: (i, 0)),
            pl.BlockSpec((blk, ncb), lambda i: (i, 0)),
            pl.BlockSpec((1, 1), lambda i: (0, 0)),
        ),
        out_shape=out_shapes,
        compiler_params=pltpu.CompilerParams(
            dimension_semantics=("arbitrary",)),
    )(embeddings, enc_w0.astype(bf), row2(enc_b0), enc_w1.astype(bf),
      row2(enc_b1), enc_w2.astype(bf), row2(enc_b2), dec_w0.astype(bf),
      row2(dec_b0), dec_w1.astype(bf), row2(dec_b1), dec_w2.astype(bf),
      row2(dec_b2), cb_t, cb_hi, cb_mid, cb_lo, cb_norm)
    return recon, idx.astype(jnp.int64), loss[0, 0]


# speedup vs baseline: 1.7188x; 1.1582x over previous
import functools

import jax
import jax.numpy as jnp
from jax.experimental import pallas as pl
from jax.experimental.pallas import tpu as pltpu

_BETA = 0.25


def _rqvae_kernel(
    x_ref, w0_ref, b0_ref, w1_ref, b1_ref, w2_ref, b2_ref,
    dw0_ref, db0_ref, dw1_ref, db1_ref, dw2_ref, db2_ref,
    cbt_ref, cbh_ref, cbm_ref, cbl_ref, cbn_ref,
    out_ref, idx_ref, loss_ref, *, n_codebooks, batch_total, n_chains):
    bf = jnp.bfloat16
    f32 = jnp.float32
    dot = lambda a, b: jnp.dot(a, b, preferred_element_type=f32)

    def chain(x):
        h = jax.nn.relu(dot(x.astype(bf), w0_ref[...]) + b0_ref[...])
        h = jax.nn.relu(dot(h.astype(bf), w1_ref[...]) + b1_ref[...])
        z = dot(h.astype(bf), w2_ref[...]) + b2_ref[...]

        rows = z.shape[0]
        k = cbt_ref.shape[2]
        residual = z
        quant = jnp.zeros_like(z)
        loss_sum = jnp.float32(0.0)
        idx_cols = []
        lane_ids = jax.lax.broadcasted_iota(jnp.int32, (rows, k), 1)
        for i in range(n_codebooks):
            score = dot(residual.astype(bf), cbt_ref[i]) + cbn_ref[i]
            idx = jnp.argmin(score, axis=1).astype(jnp.int32)
            one_hot = (lane_ids == idx[:, None]).astype(bf)
            e = ((dot(one_hot, cbh_ref[i]) + dot(one_hot, cbm_ref[i]))
                 + dot(one_hot, cbl_ref[i]))
            residual = residual - e
            loss_sum = loss_sum + jnp.sum(residual * residual)
            quant = quant + e
            idx_cols.append(idx[:, None])

        r = jax.nn.relu(dot(quant.astype(bf), dw0_ref[...]) + db0_ref[...])
        r = jax.nn.relu(dot(r.astype(bf), dw1_ref[...]) + db1_ref[...])
        recon = dot(r.astype(bf), dw2_ref[...]) + db2_ref[...]
        return recon, jnp.concatenate(idx_cols, axis=1), loss_sum

    blk = x_ref.shape[0]
    part = blk // n_chains
    loss_total = jnp.float32(0.0)
    for c in range(n_chains):
        lo, hi = c * part, (c + 1) * part
        recon_c, idx_c, loss_c = chain(x_ref[lo:hi, :])
        out_ref[lo:hi, :] = recon_c
        idx_ref[lo:hi, :] = idx_c
        loss_total = loss_total + loss_c

    @pl.when(pl.program_id(0) == 0)
    def _():
        loss_ref[...] = jnp.zeros_like(loss_ref)

    loss_ref[...] += loss_total * ((1.0 + _BETA) / batch_total)


def kernel(embeddings, enc_w0, enc_b0, enc_w1, enc_b1, enc_w2, enc_b2,
           dec_w0, dec_b0, dec_w1, dec_b1, dec_w2, dec_b2, codebooks):
    b_total, d_in = embeddings.shape
    ncb, k, e_dim = codebooks.shape
    blk = 1024 if b_total % 1024 == 0 else b_total
    grid = b_total // blk

    bf = jnp.bfloat16
    f32 = jnp.float32
    cb_hi = codebooks.astype(bf)
    rem = codebooks - cb_hi.astype(f32)
    cb_mid = rem.astype(bf)
    cb_lo = (rem - cb_mid.astype(f32)).astype(bf)
    cb_t = jnp.swapaxes(cb_hi * jnp.bfloat16(-2.0), 1, 2)
    cb_norm = jnp.sum(codebooks * codebooks, axis=2)[:, None, :]

    full = lambda shape: pl.BlockSpec(shape, lambda i: (0,) * len(shape))
    row2 = lambda v: v.reshape(1, -1)

    kern = functools.partial(_rqvae_kernel, n_codebooks=ncb,
                             batch_total=float(b_total),
                             n_chains=max(1, blk // 256))
    out_shapes = (
        jax.ShapeDtypeStruct((b_total, d_in), jnp.float32),
        jax.ShapeDtypeStruct((b_total, ncb), jnp.int32),
        jax.ShapeDtypeStruct((1, 1), jnp.float32),
    )
    recon, idx, loss = pl.pallas_call(
        kern,
        grid=(grid,),
        in_specs=[
            pl.BlockSpec((blk, d_in), lambda i: (i, 0)),
            full(enc_w0.shape), full((1, enc_b0.shape[0])),
            full(enc_w1.shape), full((1, enc_b1.shape[0])),
            full(enc_w2.shape), full((1, enc_b2.shape[0])),
            full(dec_w0.shape), full((1, dec_b0.shape[0])),
            full(dec_w1.shape), full((1, dec_b1.shape[0])),
            full(dec_w2.shape), full((1, dec_b2.shape[0])),
            full((ncb, e_dim, k)),
            full(codebooks.shape), full(codebooks.shape), full(codebooks.shape),
            full((ncb, 1, k)),
        ],
        out_specs=(
            pl.BlockSpec((blk, d_in), lambda i: (i, 0)),
            pl.BlockSpec((blk, ncb), lambda i: (i, 0)),
            pl.BlockSpec((1, 1), lambda i: (0, 0)),
        ),
        out_shape=out_shapes,
        compiler_params=pltpu.CompilerParams(
            dimension_semantics=("arbitrary",)),
    )(embeddings, enc_w0.astype(bf), row2(enc_b0), enc_w1.astype(bf),
      row2(enc_b1), enc_w2.astype(bf), row2(enc_b2), dec_w0.astype(bf),
      row2(dec_b0), dec_w1.astype(bf), row2(dec_b1), dec_w2.astype(bf),
      row2(dec_b2), cb_t, cb_hi, cb_mid, cb_lo, cb_norm)
    return recon, idx.astype(jnp.int64), loss[0, 0]
